# Initial kernel scaffold; baseline (speedup 1.0000x reference)
import functools

import jax
import jax.numpy as jnp
from jax import lax
from jax.experimental import pallas as pl
from jax.experimental.pallas import tpu as pltpu
from jax.experimental.pallas import tpu_sc as plsc

NC = 2
NS = 16
NW = NC * NS
LANES = 16
CHUNK = 80


def _matmul_kernel(x_ref, w_ref, o_ref):
    o_ref[...] = jnp.dot(x_ref[...], w_ref[...],
                         preferred_element_type=jnp.float32)


def _tc_matmul(X, W, block_rows):
    n, k = X.shape
    m = W.shape[1]
    grid = n // block_rows
    return pl.pallas_call(
        _matmul_kernel,
        grid=(grid,),
        in_specs=[
            pl.BlockSpec((block_rows, k), lambda i: (i, 0)),
            pl.BlockSpec((k, m), lambda i: (0, 0)),
        ],
        out_specs=pl.BlockSpec((block_rows, m), lambda i: (i, 0)),
        out_shape=jax.ShapeDtypeStruct((n, m), jnp.float32),
    )(X, W)


def _gates_kernel(e_ref, w_ref, b_ref, o_ref):
    logits = jnp.dot(e_ref[...], w_ref[...],
                     preferred_element_type=jnp.float32) + b_ref[...]
    o_ref[...] = jax.nn.sigmoid(logits)


def _tc_gates(E2, Wbig, bbig, block_rows):
    n = E2.shape[0]
    grid = n // block_rows
    return pl.pallas_call(
        _gates_kernel,
        grid=(grid,),
        in_specs=[
            pl.BlockSpec((block_rows, 128), lambda i: (i, 0)),
            pl.BlockSpec((128, 16), lambda i: (0, 0)),
            pl.BlockSpec((1, 16), lambda i: (0, 0)),
        ],
        out_specs=pl.BlockSpec((block_rows, 16), lambda i: (i, 0)),
        out_shape=jax.ShapeDtypeStruct((n, 16), jnp.float32),
    )(E2, Wbig, bbig)


def _mid_kernel(p_ref, b_ref, w_ref, o_ref):
    h = jax.nn.relu(p_ref[0] + p_ref[1] + b_ref[...])
    o_ref[...] = jnp.dot(h, w_ref[...], preferred_element_type=jnp.float32)


def _tc_mid(parts, b1, W2p, block_rows):
    _, n, d = parts.shape
    m = W2p.shape[1]
    grid = n // block_rows
    return pl.pallas_call(
        _mid_kernel,
        grid=(grid,),
        in_specs=[
            pl.BlockSpec((2, block_rows, d), lambda i: (0, i, 0)),
            pl.BlockSpec((1, d), lambda i: (0, 0)),
            pl.BlockSpec((d, m), lambda i: (0, 0)),
        ],
        out_specs=pl.BlockSpec((block_rows, m), lambda i: (i, 0)),
        out_shape=jax.ShapeDtypeStruct((n, m), jnp.float32),
    )(parts, b1, W2p)


def _final_kernel(p_ref, b_ref, o_ref, *, n_classes):
    x = p_ref[0] + p_ref[1]
    logits = x[:, :n_classes] + b_ref[...]
    m = jnp.max(logits, axis=1, keepdims=True)
    s = jnp.log(jnp.sum(jnp.exp(logits - m), axis=1, keepdims=True))
    o_ref[...] = logits - m - s


def _tc_final(parts, b2, block_rows):
    _, n, d = parts.shape
    n_classes = b2.shape[1]
    grid = n // block_rows
    return pl.pallas_call(
        functools.partial(_final_kernel, n_classes=n_classes),
        grid=(grid,),
        in_specs=[
            pl.BlockSpec((2, block_rows, d), lambda i: (0, i, 0)),
            pl.BlockSpec((1, n_classes), lambda i: (0, 0)),
        ],
        out_specs=pl.BlockSpec((block_rows, n_classes), lambda i: (i, 0)),
        out_shape=jax.ShapeDtypeStruct((n, n_classes), jnp.float32),
    )(parts, b2)


def _sc_aggregate(XW, A, gates, col):
    n, d = XW.shape
    ne = A.shape[1]
    ept = ne // NW
    nchunk = ept // CHUNK
    rpt = n // NS
    zrows = 125

    @functools.partial(
        pl.kernel,
        out_type=jax.ShapeDtypeStruct((NC, n, d), jnp.float32),
        mesh=plsc.VectorSubcoreMesh(core_axis_name="c", subcore_axis_name="s"),
        scratch_types=[
            pltpu.VMEM((CHUNK,), jnp.int32),
            pltpu.VMEM((CHUNK,), jnp.int32),
            pltpu.VMEM((CHUNK, 2), jnp.float32),
            pltpu.VMEM((CHUNK, d), jnp.float32),
            pltpu.VMEM((zrows, d), jnp.float32),
            pltpu.VMEM_SHARED((n, d), jnp.float32),
        ],
    )
    def k(xw_hbm, a_hbm, g_hbm, out_hbm, src_v, dst_v, g_v, rows_v, zero_v,
          acc_sh):
        cid = lax.axis_index("c")
        sid = lax.axis_index("s")
        wid = sid * NC + cid

        @pl.loop(0, zrows)
        def _(r):
            for f in range(0, d, LANES):
                zero_v[r, pl.ds(f, LANES)] = jnp.zeros((LANES,), jnp.float32)

        @pl.loop(0, rpt, step=zrows)
        def _(r0):
            pltpu.sync_copy(zero_v, acc_sh.at[pl.ds(sid * rpt + r0, zrows)])

        plsc.subcore_barrier()

        base_e = wid * ept

        @pl.loop(0, nchunk)
        def _(c):
            e0 = base_e + c * CHUNK
            pltpu.sync_copy(a_hbm.at[0, pl.ds(e0, CHUNK)], src_v)
            pltpu.sync_copy(a_hbm.at[1, pl.ds(e0, CHUNK)], dst_v)
            pltpu.sync_copy(g_hbm.at[pl.ds(e0, CHUNK)], g_v)
            pltpu.sync_copy(xw_hbm.at[src_v], rows_v)

            @pl.loop(0, CHUNK)
            def _(e):
                g = g_v[e, col]
                for f in range(0, d, LANES):
                    rows_v[e, pl.ds(f, LANES)] = rows_v[e, pl.ds(f, LANES)] * g

            pltpu.sync_copy(rows_v, acc_sh.at[dst_v], add=True)

        plsc.subcore_barrier()

        pltpu.sync_copy(acc_sh.at[pl.ds(sid * rpt, rpt)],
                        out_hbm.at[cid, pl.ds(sid * rpt, rpt)])

    return k(XW, A, gates)


def kernel(H, A, E, W1, b1, We1, be1, W2, b2, We2, be2):
    n, d_node = H.shape
    ne = A.shape[1]
    d_edge = E.shape[1]
    n_classes = W2.shape[1]
    d2 = 48

    Wcat = jnp.concatenate([We1, We2], axis=1)
    Wbig = jnp.kron(jnp.eye(8, dtype=jnp.float32), Wcat)
    bbig = jnp.tile(jnp.concatenate([be1, be2]), 8)[None, :]
    E2 = E.reshape(ne // 8, 8 * d_edge)

    gates16 = _tc_gates(E2, Wbig, bbig, block_rows=4000)
    gates = gates16.reshape(ne, 2)

    XW1 = _tc_matmul(H, W1, block_rows=2000)
    parts1 = _sc_aggregate(XW1, A, gates, col=0)

    W2p = jnp.pad(W2, ((0, 0), (0, d2 - n_classes)))
    XW2 = _tc_mid(parts1, b1[None, :], W2p, block_rows=2000)
    parts2 = _sc_aggregate(XW2, A, gates, col=1)

    return _tc_final(parts2, b2[None, :], block_rows=2000)


# speedup vs baseline: 3.4545x; 3.4545x over previous
import functools

import jax
import jax.numpy as jnp
from jax import lax
from jax.experimental import pallas as pl
from jax.experimental.pallas import tpu as pltpu
from jax.experimental.pallas import tpu_sc as plsc

NC = 2
NS = 16
NW = NC * NS
LANES = 16
CHUNK = 80


def _matmul_kernel(x_ref, w_ref, o_ref):
    o_ref[...] = jnp.dot(x_ref[...], w_ref[...],
                         preferred_element_type=jnp.float32)


def _tc_matmul(X, W, block_rows):
    n, k = X.shape
    m = W.shape[1]
    grid = n // block_rows
    return pl.pallas_call(
        _matmul_kernel,
        grid=(grid,),
        in_specs=[
            pl.BlockSpec((block_rows, k), lambda i: (i, 0)),
            pl.BlockSpec((k, m), lambda i: (0, 0)),
        ],
        out_specs=pl.BlockSpec((block_rows, m), lambda i: (i, 0)),
        out_shape=jax.ShapeDtypeStruct((n, m), jnp.float32),
    )(X, W)


def _gates_kernel(e_ref, w_ref, b_ref, o1_ref, o2_ref):
    logits = jnp.dot(e_ref[...], w_ref[...],
                     preferred_element_type=jnp.float32) + b_ref[...]
    g = jax.nn.sigmoid(logits)
    o1_ref[...] = g[:, :8]
    o2_ref[...] = g[:, 8:]


def _tc_gates(E2, Wbig, bbig, block_rows):
    n = E2.shape[0]
    grid = n // block_rows
    out = jax.ShapeDtypeStruct((n, 8), jnp.float32)
    return pl.pallas_call(
        _gates_kernel,
        grid=(grid,),
        in_specs=[
            pl.BlockSpec((block_rows, 128), lambda i: (i, 0)),
            pl.BlockSpec((128, 16), lambda i: (0, 0)),
            pl.BlockSpec((1, 16), lambda i: (0, 0)),
        ],
        out_specs=[pl.BlockSpec((block_rows, 8), lambda i: (i, 0))] * 2,
        out_shape=[out, out],
    )(E2, Wbig, bbig)


def _mid_kernel(p_ref, b_ref, w_ref, o_ref):
    h = jax.nn.relu(p_ref[0] + p_ref[1] + b_ref[...])
    o_ref[...] = jnp.dot(h, w_ref[...], preferred_element_type=jnp.float32)


def _tc_mid(parts, b1, W2p, block_rows):
    _, n, d = parts.shape
    m = W2p.shape[1]
    grid = n // block_rows
    return pl.pallas_call(
        _mid_kernel,
        grid=(grid,),
        in_specs=[
            pl.BlockSpec((2, block_rows, d), lambda i: (0, i, 0)),
            pl.BlockSpec((1, d), lambda i: (0, 0)),
            pl.BlockSpec((d, m), lambda i: (0, 0)),
        ],
        out_specs=pl.BlockSpec((block_rows, m), lambda i: (i, 0)),
        out_shape=jax.ShapeDtypeStruct((n, m), jnp.float32),
    )(parts, b1, W2p)


def _final_kernel(p_ref, b_ref, o_ref, *, n_classes):
    x = p_ref[0] + p_ref[1]
    logits = x[:, :n_classes] + b_ref[...]
    m = jnp.max(logits, axis=1, keepdims=True)
    s = jnp.log(jnp.sum(jnp.exp(logits - m), axis=1, keepdims=True))
    o_ref[...] = logits - m - s


def _tc_final(parts, b2, block_rows):
    _, n, d = parts.shape
    n_classes = b2.shape[1]
    grid = n // block_rows
    return pl.pallas_call(
        functools.partial(_final_kernel, n_classes=n_classes),
        grid=(grid,),
        in_specs=[
            pl.BlockSpec((2, block_rows, d), lambda i: (0, i, 0)),
            pl.BlockSpec((1, n_classes), lambda i: (0, 0)),
        ],
        out_specs=pl.BlockSpec((block_rows, n_classes), lambda i: (i, 0)),
        out_shape=jax.ShapeDtypeStruct((n, n_classes), jnp.float32),
    )(parts, b2)


def _sc_aggregate(XW, src, dst, gate, n_out):
    d = XW.shape[1]
    ne = src.shape[0]
    ept = ne // NW
    nchunk = ept // CHUNK
    rpt = n_out // NS
    zrows = 128

    @functools.partial(
        pl.kernel,
        out_type=jax.ShapeDtypeStruct((NC, n_out, d), jnp.float32),
        mesh=plsc.VectorSubcoreMesh(core_axis_name="c", subcore_axis_name="s"),
        scratch_types=[
            pltpu.VMEM((CHUNK,), jnp.int32),
            pltpu.VMEM((CHUNK,), jnp.int32),
            pltpu.VMEM((CHUNK,), jnp.float32),
            pltpu.VMEM((CHUNK, d), jnp.float32),
            pltpu.VMEM((zrows, d), jnp.float32),
            pltpu.VMEM_SHARED((n_out, d), jnp.float32),
        ],
    )
    def k(xw_hbm, src_hbm, dst_hbm, g_hbm, out_hbm, src_v, dst_v, g_v, rows_v, zero_v,
          acc_sh):
        cid = lax.axis_index("c")
        sid = lax.axis_index("s")
        wid = sid * NC + cid

        @pl.loop(0, zrows)
        def _(r):
            for f in range(0, d, LANES):
                zero_v[r, pl.ds(f, LANES)] = jnp.zeros((LANES,), jnp.float32)

        @pl.loop(0, rpt, step=zrows)
        def _(r0):
            pltpu.sync_copy(zero_v, acc_sh.at[pl.ds(sid * rpt + r0, zrows)])

        plsc.subcore_barrier()

        base_e = wid * ept

        @pl.loop(0, nchunk)
        def _(c):
            e0 = base_e + c * CHUNK
            pltpu.sync_copy(src_hbm.at[pl.ds(e0, CHUNK)], src_v)
            pltpu.sync_copy(dst_hbm.at[pl.ds(e0, CHUNK)], dst_v)
            pltpu.sync_copy(g_hbm.at[pl.ds(e0, CHUNK)], g_v)
            pltpu.sync_copy(xw_hbm.at[src_v], rows_v)

            @pl.loop(0, CHUNK, step=LANES)
            def _(eg):
                gvec = g_v[pl.ds(eg, LANES)]
                for j in range(LANES):
                    g = gvec[j]
                    for f in range(0, d, LANES):
                        rows_v[eg + j, pl.ds(f, LANES)] = (
                            rows_v[eg + j, pl.ds(f, LANES)] * g)

            pltpu.sync_copy(rows_v, acc_sh.at[dst_v], add=True)

        plsc.subcore_barrier()

        pltpu.sync_copy(acc_sh.at[pl.ds(sid * rpt, rpt)],
                        out_hbm.at[cid, pl.ds(sid * rpt, rpt)])

    return k(XW, src, dst, gate)


_N_PAD = 10240


def kernel(H, A, E, W1, b1, We1, be1, W2, b2, We2, be2):
    n, d_node = H.shape
    ne = A.shape[1]
    d_edge = E.shape[1]
    n_classes = W2.shape[1]
    d2 = 128

    eye8 = jnp.eye(8, dtype=jnp.float32)
    Wbig = jnp.concatenate(
        [jnp.kron(eye8, We1), jnp.kron(eye8, We2)], axis=1)
    bbig = jnp.concatenate(
        [jnp.tile(be1, 8), jnp.tile(be2, 8)])[None, :]
    E2 = E.reshape(ne // 8, 8 * d_edge)

    g1_8, g2_8 = _tc_gates(E2, Wbig, bbig, block_rows=4000)
    gate1 = g1_8.reshape(ne)
    gate2 = g2_8.reshape(ne)

    src = A[0]
    dst = A[1]
    XW1 = _tc_matmul(H, W1, block_rows=2000)
    parts1 = _sc_aggregate(XW1, src, dst, gate1, _N_PAD)

    W2p = jnp.pad(W2, ((0, 0), (0, d2 - n_classes)))
    XW2 = _tc_mid(parts1, b1[None, :], W2p, block_rows=2048)
    parts2 = _sc_aggregate(XW2, src, dst, gate2, _N_PAD)

    out = _tc_final(parts2, b2[None, :], block_rows=2048)
    return out[:n]


# speedup vs baseline: 4.6199x; 1.3374x over previous
import functools

import jax
import jax.numpy as jnp
from jax import lax
from jax.experimental import pallas as pl
from jax.experimental.pallas import tpu as pltpu
from jax.experimental.pallas import tpu_sc as plsc

NC = 2
NS = 16
NW = NC * NS
LANES = 16
CHUNK = 80


def _matmul_kernel(x_ref, w_ref, o_ref):
    o_ref[...] = jnp.dot(x_ref[...], w_ref[...],
                         preferred_element_type=jnp.float32)


def _tc_matmul(X, W, block_rows):
    n, k = X.shape
    m = W.shape[1]
    grid = n // block_rows
    return pl.pallas_call(
        _matmul_kernel,
        grid=(grid,),
        in_specs=[
            pl.BlockSpec((block_rows, k), lambda i: (i, 0)),
            pl.BlockSpec((k, m), lambda i: (0, 0)),
        ],
        out_specs=pl.BlockSpec((block_rows, m), lambda i: (i, 0)),
        out_shape=jax.ShapeDtypeStruct((n, m), jnp.float32),
    )(X, W)


def _gates_kernel(e_ref, w_ref, b_ref, o1_ref, o2_ref):
    logits = jnp.dot(e_ref[...], w_ref[...],
                     preferred_element_type=jnp.float32) + b_ref[...]
    g = jax.nn.sigmoid(logits)
    o1_ref[...] = g[:, :8]
    o2_ref[...] = g[:, 8:]


def _tc_gates(E2, Wbig, bbig, block_rows):
    n = E2.shape[0]
    grid = n // block_rows
    out = jax.ShapeDtypeStruct((n, 8), jnp.float32)
    return pl.pallas_call(
        _gates_kernel,
        grid=(grid,),
        in_specs=[
            pl.BlockSpec((block_rows, 128), lambda i: (i, 0)),
            pl.BlockSpec((128, 16), lambda i: (0, 0)),
            pl.BlockSpec((1, 16), lambda i: (0, 0)),
        ],
        out_specs=[pl.BlockSpec((block_rows, 8), lambda i: (i, 0))] * 2,
        out_shape=[out, out],
    )(E2, Wbig, bbig)


def _mid_kernel(p_ref, b_ref, w_ref, o_ref):
    h = jax.nn.relu(p_ref[0] + p_ref[1] + b_ref[...])
    o_ref[...] = jnp.dot(h, w_ref[...], preferred_element_type=jnp.float32)


def _tc_mid(parts, b1, W2p, block_rows):
    _, n, d = parts.shape
    m = W2p.shape[1]
    grid = n // block_rows
    return pl.pallas_call(
        _mid_kernel,
        grid=(grid,),
        in_specs=[
            pl.BlockSpec((2, block_rows, d), lambda i: (0, i, 0)),
            pl.BlockSpec((1, d), lambda i: (0, 0)),
            pl.BlockSpec((d, m), lambda i: (0, 0)),
        ],
        out_specs=pl.BlockSpec((block_rows, m), lambda i: (i, 0)),
        out_shape=jax.ShapeDtypeStruct((n, m), jnp.float32),
    )(parts, b1, W2p)


def _final_kernel(p_ref, b_ref, o_ref, *, n_classes):
    x = p_ref[0] + p_ref[1]
    logits = x[:, :n_classes] + b_ref[...]
    m = jnp.max(logits, axis=1, keepdims=True)
    s = jnp.log(jnp.sum(jnp.exp(logits - m), axis=1, keepdims=True))
    o_ref[...] = logits - m - s


def _tc_final(parts, b2, block_rows):
    _, n, d = parts.shape
    n_classes = b2.shape[1]
    grid = n // block_rows
    return pl.pallas_call(
        functools.partial(_final_kernel, n_classes=n_classes),
        grid=(grid,),
        in_specs=[
            pl.BlockSpec((2, block_rows, d), lambda i: (0, i, 0)),
            pl.BlockSpec((1, n_classes), lambda i: (0, 0)),
        ],
        out_specs=pl.BlockSpec((block_rows, n_classes), lambda i: (i, 0)),
        out_shape=jax.ShapeDtypeStruct((n, n_classes), jnp.float32),
    )(parts, b2)


def _sc_aggregate(XW, src, dst, gate, n_out):
    d = XW.shape[1]
    ne = src.shape[0]
    tot_chunks = ne // CHUNK
    rpt = n_out // NS
    zrows = rpt // 4

    @functools.partial(
        pl.kernel,
        out_type=jax.ShapeDtypeStruct((NC, n_out, d), jnp.float32),
        mesh=plsc.VectorSubcoreMesh(core_axis_name="c", subcore_axis_name="s"),
        scratch_types=[
            pltpu.VMEM((CHUNK,), jnp.int32),
            pltpu.VMEM((CHUNK,), jnp.int32),
            pltpu.VMEM((CHUNK,), jnp.int32),
            pltpu.VMEM((CHUNK,), jnp.int32),
            pltpu.VMEM((CHUNK,), jnp.float32),
            pltpu.VMEM((CHUNK,), jnp.float32),
            pltpu.VMEM((CHUNK, d), jnp.float32),
            pltpu.VMEM((CHUNK, d), jnp.float32),
            pltpu.VMEM((zrows, d), jnp.float32),
            pltpu.VMEM_SHARED((n_out, d), jnp.float32),
            pltpu.SemaphoreType.DMA,
            pltpu.SemaphoreType.DMA,
        ],
    )
    def k(xw_hbm, src_hbm, dst_hbm, g_hbm, out_hbm, src0, src1, dst0, dst1,
          g0, g1, rows0, rows1, zero_v, acc_sh, sem0, sem1):
        cid = lax.axis_index("c")
        sid = lax.axis_index("s")
        wid = sid * NC + cid

        def load_idx(t, src_b, dst_b, g_b):
            e0 = (t * NW + wid) * CHUNK
            pltpu.sync_copy(src_hbm.at[pl.ds(e0, CHUNK)], src_b)
            pltpu.sync_copy(dst_hbm.at[pl.ds(e0, CHUNK)], dst_b)
            pltpu.sync_copy(g_hbm.at[pl.ds(e0, CHUNK)], g_b)

        def gather_start(src_b, rows_b, sem):
            pltpu.async_copy(xw_hbm.at[src_b], rows_b, sem)

        def gather_wait(src_b, rows_b, sem):
            pltpu.make_async_copy(xw_hbm.at[src_b], rows_b, sem).wait()

        def scale(rows_b, g_b):
            @plsc.parallel_loop(0, CHUNK, step=LANES, unroll=2)
            def _(eg):
                gvec = g_b[pl.ds(eg, LANES)]
                for j in range(LANES):
                    g = gvec[j]
                    for f in range(0, d, LANES):
                        rows_b[eg + j, pl.ds(f, LANES)] = (
                            rows_b[eg + j, pl.ds(f, LANES)] * g)

        base_ct = tot_chunks // NW
        nct = base_ct + jnp.where(wid < tot_chunks % NW, 1, 0)
        npairs = nct // 2

        load_idx(0, src0, dst0, g0)
        gather_start(src0, rows0, sem0)

        @pl.loop(0, zrows)
        def _(r):
            for f in range(0, d, LANES):
                zero_v[r, pl.ds(f, LANES)] = jnp.zeros((LANES,), jnp.float32)

        @pl.loop(0, rpt, step=zrows)
        def _(r0):
            pltpu.sync_copy(zero_v, acc_sh.at[pl.ds(sid * rpt + r0, zrows)])

        plsc.subcore_barrier()

        @pl.loop(0, npairs)
        def _(t):
            load_idx(2 * t + 1, src1, dst1, g1)
            gather_start(src1, rows1, sem1)
            gather_wait(src0, rows0, sem0)
            scale(rows0, g0)
            pltpu.sync_copy(rows0, acc_sh.at[dst0], add=True)

            @pl.when(2 * t + 2 < nct)
            def _():
                load_idx(2 * t + 2, src0, dst0, g0)
                gather_start(src0, rows0, sem0)

            gather_wait(src1, rows1, sem1)
            scale(rows1, g1)
            pltpu.sync_copy(rows1, acc_sh.at[dst1], add=True)

        @pl.when(nct % 2 == 1)
        def _():
            gather_wait(src0, rows0, sem0)
            scale(rows0, g0)
            pltpu.sync_copy(rows0, acc_sh.at[dst0], add=True)

        plsc.subcore_barrier()

        pltpu.sync_copy(acc_sh.at[pl.ds(sid * rpt, rpt)],
                        out_hbm.at[cid, pl.ds(sid * rpt, rpt)])

    return k(XW, src, dst, gate)


_N_PAD = 10112


def kernel(H, A, E, W1, b1, We1, be1, W2, b2, We2, be2):
    n, d_node = H.shape
    ne = A.shape[1]
    d_edge = E.shape[1]
    n_classes = W2.shape[1]
    d2 = 128

    eye8 = jnp.eye(8, dtype=jnp.float32)
    Wbig = jnp.concatenate(
        [jnp.kron(eye8, We1), jnp.kron(eye8, We2)], axis=1)
    bbig = jnp.concatenate(
        [jnp.tile(be1, 8), jnp.tile(be2, 8)])[None, :]
    E2 = E.reshape(ne // 8, 8 * d_edge)

    g1_8, g2_8 = _tc_gates(E2, Wbig, bbig, block_rows=4000)
    gate1 = g1_8.reshape(ne)
    gate2 = g2_8.reshape(ne)

    src = A[0]
    dst = A[1]
    XW1 = _tc_matmul(H, W1, block_rows=2000)
    parts1 = _sc_aggregate(XW1, src, dst, gate1, _N_PAD)

    W2p = jnp.pad(W2, ((0, 0), (0, d2 - n_classes)))
    XW2 = _tc_mid(parts1, b1[None, :], W2p, block_rows=1264)
    parts2 = _sc_aggregate(XW2, src, dst, gate2, _N_PAD)

    out = _tc_final(parts2, b2[None, :], block_rows=1264)
    return out[:n]


# speedup vs baseline: 4.7178x; 1.0212x over previous
import functools

import jax
import jax.numpy as jnp
from jax import lax
from jax.experimental import pallas as pl
from jax.experimental.pallas import tpu as pltpu
from jax.experimental.pallas import tpu_sc as plsc

NC = 2
NS = 16
NW = NC * NS
LANES = 16
CHUNK = 80


def _matmul_kernel(x_ref, w_ref, o_ref):
    o_ref[...] = jnp.dot(x_ref[...], w_ref[...],
                         preferred_element_type=jnp.float32)


def _tc_matmul(X, W, block_rows):
    n, k = X.shape
    m = W.shape[1]
    grid = n // block_rows
    return pl.pallas_call(
        _matmul_kernel,
        grid=(grid,),
        in_specs=[
            pl.BlockSpec((block_rows, k), lambda i: (i, 0)),
            pl.BlockSpec((k, m), lambda i: (0, 0)),
        ],
        out_specs=pl.BlockSpec((block_rows, m), lambda i: (i, 0)),
        out_shape=jax.ShapeDtypeStruct((n, m), jnp.float32),
    )(X, W)


def _gates_kernel(e_ref, w_ref, b_ref, o1_ref, o2_ref):
    logits = jnp.dot(e_ref[...], w_ref[...],
                     preferred_element_type=jnp.float32) + b_ref[...]
    g = jax.nn.sigmoid(logits)
    o1_ref[...] = g[:, :8]
    o2_ref[...] = g[:, 8:]


def _tc_gates(E2, Wbig, bbig, block_rows):
    n = E2.shape[0]
    grid = n // block_rows
    out = jax.ShapeDtypeStruct((n, 8), jnp.float32)
    return pl.pallas_call(
        _gates_kernel,
        grid=(grid,),
        in_specs=[
            pl.BlockSpec((block_rows, 128), lambda i: (i, 0)),
            pl.BlockSpec((128, 16), lambda i: (0, 0)),
            pl.BlockSpec((1, 16), lambda i: (0, 0)),
        ],
        out_specs=[pl.BlockSpec((block_rows, 8), lambda i: (i, 0))] * 2,
        out_shape=[out, out],
    )(E2, Wbig, bbig)


def _mid_kernel(p_ref, b_ref, w_ref, o_ref):
    h = jax.nn.relu(p_ref[0] + p_ref[1] + b_ref[...])
    o_ref[...] = jnp.dot(h, w_ref[...], preferred_element_type=jnp.float32)


def _tc_mid(parts, b1, W2p, block_rows):
    _, n, d = parts.shape
    m = W2p.shape[1]
    grid = n // block_rows
    return pl.pallas_call(
        _mid_kernel,
        grid=(grid,),
        in_specs=[
            pl.BlockSpec((2, block_rows, d), lambda i: (0, i, 0)),
            pl.BlockSpec((1, d), lambda i: (0, 0)),
            pl.BlockSpec((d, m), lambda i: (0, 0)),
        ],
        out_specs=pl.BlockSpec((block_rows, m), lambda i: (i, 0)),
        out_shape=jax.ShapeDtypeStruct((n, m), jnp.float32),
    )(parts, b1, W2p)


def _final_kernel(p_ref, b_ref, o_ref, *, n_classes):
    x = p_ref[0] + p_ref[1]
    logits = x[:, :n_classes] + b_ref[...]
    m = jnp.max(logits, axis=1, keepdims=True)
    s = jnp.log(jnp.sum(jnp.exp(logits - m), axis=1, keepdims=True))
    o_ref[...] = logits - m - s


def _tc_final(parts, b2, block_rows):
    _, n, d = parts.shape
    n_classes = b2.shape[1]
    grid = n // block_rows
    return pl.pallas_call(
        functools.partial(_final_kernel, n_classes=n_classes),
        grid=(grid,),
        in_specs=[
            pl.BlockSpec((2, block_rows, d), lambda i: (0, i, 0)),
            pl.BlockSpec((1, n_classes), lambda i: (0, 0)),
        ],
        out_specs=pl.BlockSpec((block_rows, n_classes), lambda i: (i, 0)),
        out_shape=jax.ShapeDtypeStruct((n, n_classes), jnp.float32),
    )(parts, b2)


def _sc_aggregate(XW, src, dst, gate, n_out):
    d = XW.shape[1]
    ne = src.shape[0]
    tot_chunks = ne // CHUNK
    rpt = n_out // NS
    zrows = rpt // 4

    @functools.partial(
        pl.kernel,
        out_type=jax.ShapeDtypeStruct((NC, n_out, d), jnp.float32),
        mesh=plsc.VectorSubcoreMesh(core_axis_name="c", subcore_axis_name="s"),
        scratch_types=[
            pltpu.VMEM((CHUNK,), jnp.int32),
            pltpu.VMEM((CHUNK,), jnp.int32),
            pltpu.VMEM((CHUNK,), jnp.int32),
            pltpu.VMEM((CHUNK,), jnp.int32),
            pltpu.VMEM((CHUNK,), jnp.float32),
            pltpu.VMEM((CHUNK,), jnp.float32),
            pltpu.VMEM((CHUNK, d), jnp.float32),
            pltpu.VMEM((CHUNK, d), jnp.float32),
            pltpu.VMEM((zrows, d), jnp.float32),
            pltpu.VMEM_SHARED((n_out, d), jnp.float32),
            pltpu.SemaphoreType.DMA,
            pltpu.SemaphoreType.DMA,
            pltpu.SemaphoreType.DMA,
            pltpu.SemaphoreType.DMA,
        ],
    )
    def k(xw_hbm, src_hbm, dst_hbm, g_hbm, out_hbm, src0, src1, dst0, dst1,
          g0, g1, rows0, rows1, zero_v, acc_sh, sem0, sem1, sems0, sems1):
        cid = lax.axis_index("c")
        sid = lax.axis_index("s")
        wid = sid * NC + cid

        def load_idx(t, src_b, dst_b, g_b):
            e0 = (t * NW + wid) * CHUNK
            pltpu.sync_copy(src_hbm.at[pl.ds(e0, CHUNK)], src_b)
            pltpu.sync_copy(dst_hbm.at[pl.ds(e0, CHUNK)], dst_b)
            pltpu.sync_copy(g_hbm.at[pl.ds(e0, CHUNK)], g_b)

        def gather_start(src_b, rows_b, sem):
            pltpu.async_copy(xw_hbm.at[src_b], rows_b, sem)

        def gather_wait(src_b, rows_b, sem):
            pltpu.make_async_copy(xw_hbm.at[src_b], rows_b, sem).wait()

        def scatter_start(rows_b, dst_b, sem):
            pltpu.async_copy(rows_b, acc_sh.at[dst_b], sem, add=True)

        def scatter_wait(rows_b, dst_b, sem):
            pltpu.make_async_copy(rows_b, acc_sh.at[dst_b], sem).wait()

        def scale(rows_b, g_b):
            @plsc.parallel_loop(0, CHUNK, step=LANES, unroll=4)
            def _(eg):
                gvec = g_b[pl.ds(eg, LANES)]
                for j in range(LANES):
                    g = gvec[j]
                    for f in range(0, d, LANES):
                        rows_b[eg + j, pl.ds(f, LANES)] = (
                            rows_b[eg + j, pl.ds(f, LANES)] * g)

        nct = tot_chunks // NW
        npairs = nct // 2

        load_idx(0, src0, dst0, g0)
        gather_start(src0, rows0, sem0)
        load_idx(1, src1, dst1, g1)
        gather_start(src1, rows1, sem1)

        @pl.loop(0, zrows)
        def _(r):
            for f in range(0, d, LANES):
                zero_v[r, pl.ds(f, LANES)] = jnp.zeros((LANES,), jnp.float32)

        @pl.loop(0, rpt, step=zrows)
        def _(r0):
            pltpu.sync_copy(zero_v, acc_sh.at[pl.ds(sid * rpt + r0, zrows)])

        plsc.subcore_barrier()

        @pl.loop(0, npairs)
        def _(t):
            gather_wait(src0, rows0, sem0)
            scale(rows0, g0)
            scatter_start(rows0, dst0, sems0)

            gather_wait(src1, rows1, sem1)
            scale(rows1, g1)
            scatter_start(rows1, dst1, sems1)

            @pl.when(2 * t + 2 < nct)
            def _():
                scatter_wait(rows0, dst0, sems0)
                load_idx(2 * t + 2, src0, dst0, g0)
                gather_start(src0, rows0, sem0)

            @pl.when(2 * t + 3 < nct)
            def _():
                scatter_wait(rows1, dst1, sems1)
                load_idx(2 * t + 3, src1, dst1, g1)
                gather_start(src1, rows1, sem1)

            @pl.when(2 * t + 2 >= nct)
            def _():
                scatter_wait(rows0, dst0, sems0)

            @pl.when(2 * t + 3 >= nct)
            def _():
                scatter_wait(rows1, dst1, sems1)

        @pl.when(nct % 2 == 1)
        def _():
            gather_wait(src0, rows0, sem0)
            scale(rows0, g0)
            scatter_start(rows0, dst0, sems0)
            scatter_wait(rows0, dst0, sems0)

        plsc.subcore_barrier()

        pltpu.sync_copy(acc_sh.at[pl.ds(sid * rpt, rpt)],
                        out_hbm.at[cid, pl.ds(sid * rpt, rpt)])

    return k(XW, src, dst, gate)


_N_PAD = 10112


def kernel(H, A, E, W1, b1, We1, be1, W2, b2, We2, be2):
    n, d_node = H.shape
    ne = A.shape[1]
    d_edge = E.shape[1]
    n_classes = W2.shape[1]
    d2 = 128

    eye8 = jnp.eye(8, dtype=jnp.float32)
    Wbig = jnp.concatenate(
        [jnp.kron(eye8, We1), jnp.kron(eye8, We2)], axis=1)
    bbig = jnp.concatenate(
        [jnp.tile(be1, 8), jnp.tile(be2, 8)])[None, :]
    E2 = E.reshape(ne // 8, 8 * d_edge)

    g1_8, g2_8 = _tc_gates(E2, Wbig, bbig, block_rows=4000)
    gate1 = g1_8.reshape(ne)
    gate2 = g2_8.reshape(ne)

    src = A[0]
    dst = A[1]
    XW1 = _tc_matmul(H, W1, block_rows=2000)
    parts1 = _sc_aggregate(XW1, src, dst, gate1, _N_PAD)

    W2p = jnp.pad(W2, ((0, 0), (0, d2 - n_classes)))
    XW2 = _tc_mid(parts1, b1[None, :], W2p, block_rows=1264)
    parts2 = _sc_aggregate(XW2, src, dst, gate2, _N_PAD)

    out = _tc_final(parts2, b2[None, :], block_rows=1264)
    return out[:n]


# speedup vs baseline: 4.7327x; 1.0032x over previous
import functools

import jax
import jax.numpy as jnp
from jax import lax
from jax.experimental import pallas as pl
from jax.experimental.pallas import tpu as pltpu
from jax.experimental.pallas import tpu_sc as plsc

NC = 2
NS = 16
NW = NC * NS
LANES = 16
CHUNK = 80


def _matmul_kernel(x_ref, w_ref, o_ref):
    o_ref[...] = jnp.dot(x_ref[...], w_ref[...],
                         preferred_element_type=jnp.float32)


def _tc_matmul(X, W, block_rows):
    n, k = X.shape
    m = W.shape[1]
    grid = n // block_rows
    return pl.pallas_call(
        _matmul_kernel,
        grid=(grid,),
        in_specs=[
            pl.BlockSpec((block_rows, k), lambda i: (i, 0)),
            pl.BlockSpec((k, m), lambda i: (0, 0)),
        ],
        out_specs=pl.BlockSpec((block_rows, m), lambda i: (i, 0)),
        out_shape=jax.ShapeDtypeStruct((n, m), jnp.float32),
    )(X, W)


def _gates_kernel(e_ref, w_ref, b_ref, o1_ref, o2_ref):
    logits = jnp.dot(e_ref[...], w_ref[...],
                     preferred_element_type=jnp.float32) + b_ref[...]
    g = jax.nn.sigmoid(logits)
    o1_ref[...] = g[:, :8]
    o2_ref[...] = g[:, 8:]


def _tc_gates(E2, Wbig, bbig, block_rows):
    n = E2.shape[0]
    grid = n // block_rows
    out = jax.ShapeDtypeStruct((n, 8), jnp.float32)
    return pl.pallas_call(
        _gates_kernel,
        grid=(grid,),
        in_specs=[
            pl.BlockSpec((block_rows, 128), lambda i: (i, 0)),
            pl.BlockSpec((128, 16), lambda i: (0, 0)),
            pl.BlockSpec((1, 16), lambda i: (0, 0)),
        ],
        out_specs=[pl.BlockSpec((block_rows, 8), lambda i: (i, 0))] * 2,
        out_shape=[out, out],
    )(E2, Wbig, bbig)


def _mid_kernel(p_ref, b_ref, w_ref, o_ref):
    h = jax.nn.relu(p_ref[0] + p_ref[1] + b_ref[...])
    o_ref[...] = jnp.dot(h, w_ref[...], preferred_element_type=jnp.float32)


def _tc_mid(parts, b1, W2p, block_rows):
    _, n, d = parts.shape
    m = W2p.shape[1]
    grid = n // block_rows
    return pl.pallas_call(
        _mid_kernel,
        grid=(grid,),
        in_specs=[
            pl.BlockSpec((2, block_rows, d), lambda i: (0, i, 0)),
            pl.BlockSpec((1, d), lambda i: (0, 0)),
            pl.BlockSpec((d, m), lambda i: (0, 0)),
        ],
        out_specs=pl.BlockSpec((block_rows, m), lambda i: (i, 0)),
        out_shape=jax.ShapeDtypeStruct((n, m), jnp.float32),
    )(parts, b1, W2p)


def _final_kernel(p_ref, b_ref, o_ref, *, n_classes):
    x = p_ref[0] + p_ref[1]
    logits = x[:, :n_classes] + b_ref[...]
    m = jnp.max(logits, axis=1, keepdims=True)
    s = jnp.log(jnp.sum(jnp.exp(logits - m), axis=1, keepdims=True))
    o_ref[...] = logits - m - s


def _tc_final(parts, b2, block_rows):
    _, n, d = parts.shape
    n_classes = b2.shape[1]
    grid = n // block_rows
    return pl.pallas_call(
        functools.partial(_final_kernel, n_classes=n_classes),
        grid=(grid,),
        in_specs=[
            pl.BlockSpec((2, block_rows, d), lambda i: (0, i, 0)),
            pl.BlockSpec((1, n_classes), lambda i: (0, 0)),
        ],
        out_specs=pl.BlockSpec((block_rows, n_classes), lambda i: (i, 0)),
        out_shape=jax.ShapeDtypeStruct((n, n_classes), jnp.float32),
    )(parts, b2)


def _sc_aggregate(XW, src, dst, gate, n_out):
    d = XW.shape[1]
    ne = src.shape[0]
    tot_chunks = ne // CHUNK
    rpt = n_out // NS
    zrows = rpt // 4

    @functools.partial(
        pl.kernel,
        out_type=jax.ShapeDtypeStruct((NC, n_out, d), jnp.float32),
        mesh=plsc.VectorSubcoreMesh(core_axis_name="c", subcore_axis_name="s"),
        scratch_types=[
            pltpu.VMEM((CHUNK,), jnp.int32),
            pltpu.VMEM((CHUNK,), jnp.int32),
            pltpu.VMEM((CHUNK,), jnp.int32),
            pltpu.VMEM((CHUNK,), jnp.int32),
            pltpu.VMEM((CHUNK,), jnp.float32),
            pltpu.VMEM((CHUNK,), jnp.float32),
            pltpu.VMEM((CHUNK, d), jnp.float32),
            pltpu.VMEM((CHUNK, d), jnp.float32),
            pltpu.VMEM((zrows, d), jnp.float32),
            pltpu.VMEM_SHARED((n_out, d), jnp.float32),
            pltpu.SemaphoreType.DMA,
            pltpu.SemaphoreType.DMA,
            pltpu.SemaphoreType.DMA,
            pltpu.SemaphoreType.DMA,
        ],
    )
    def k(xw_hbm, src_hbm, dst_hbm, g_hbm, out_hbm, src0, src1, dst0, dst1,
          g0, g1, rows0, rows1, zero_v, acc_sh, sem0, sem1, sems0, sems1):
        cid = lax.axis_index("c")
        sid = lax.axis_index("s")
        wid = sid * NC + cid

        def load_idx(t, src_b, dst_b, g_b):
            e0 = (t * NW + wid) * CHUNK
            pltpu.sync_copy(src_hbm.at[pl.ds(e0, CHUNK)], src_b)
            pltpu.sync_copy(dst_hbm.at[pl.ds(e0, CHUNK)], dst_b)
            pltpu.sync_copy(g_hbm.at[pl.ds(e0, CHUNK)], g_b)

        def gather_start(src_b, rows_b, sem):
            pltpu.async_copy(xw_hbm.at[src_b], rows_b, sem)

        def gather_wait(src_b, rows_b, sem):
            pltpu.make_async_copy(xw_hbm.at[src_b], rows_b, sem).wait()

        def scatter_start(rows_b, dst_b, sem):
            pltpu.async_copy(rows_b, acc_sh.at[dst_b], sem, add=True)

        def scatter_wait(rows_b, dst_b, sem):
            pltpu.make_async_copy(rows_b, acc_sh.at[dst_b], sem).wait()

        def scale(rows_b, g_b):
            @plsc.parallel_loop(0, CHUNK, step=LANES, unroll=2)
            def _(eg):
                gvec = g_b[pl.ds(eg, LANES)]
                for j in range(LANES):
                    gb = gvec[jnp.full((LANES,), j, jnp.int32)]
                    for f in range(0, d, LANES):
                        rows_b[eg + j, pl.ds(f, LANES)] = (
                            rows_b[eg + j, pl.ds(f, LANES)] * gb)

        nct = tot_chunks // NW
        npairs = nct // 2

        load_idx(0, src0, dst0, g0)
        gather_start(src0, rows0, sem0)
        load_idx(1, src1, dst1, g1)
        gather_start(src1, rows1, sem1)

        @pl.loop(0, zrows)
        def _(r):
            for f in range(0, d, LANES):
                zero_v[r, pl.ds(f, LANES)] = jnp.zeros((LANES,), jnp.float32)

        @pl.loop(0, rpt, step=zrows)
        def _(r0):
            pltpu.sync_copy(zero_v, acc_sh.at[pl.ds(sid * rpt + r0, zrows)])

        plsc.subcore_barrier()

        @pl.loop(0, npairs)
        def _(t):
            gather_wait(src0, rows0, sem0)
            scale(rows0, g0)
            scatter_start(rows0, dst0, sems0)

            gather_wait(src1, rows1, sem1)
            scale(rows1, g1)
            scatter_start(rows1, dst1, sems1)

            @pl.when(2 * t + 2 < nct)
            def _():
                scatter_wait(rows0, dst0, sems0)
                load_idx(2 * t + 2, src0, dst0, g0)
                gather_start(src0, rows0, sem0)

            @pl.when(2 * t + 3 < nct)
            def _():
                scatter_wait(rows1, dst1, sems1)
                load_idx(2 * t + 3, src1, dst1, g1)
                gather_start(src1, rows1, sem1)

            @pl.when(2 * t + 2 >= nct)
            def _():
                scatter_wait(rows0, dst0, sems0)

            @pl.when(2 * t + 3 >= nct)
            def _():
                scatter_wait(rows1, dst1, sems1)

        @pl.when(nct % 2 == 1)
        def _():
            gather_wait(src0, rows0, sem0)
            scale(rows0, g0)
            scatter_start(rows0, dst0, sems0)
            scatter_wait(rows0, dst0, sems0)

        plsc.subcore_barrier()

        pltpu.sync_copy(acc_sh.at[pl.ds(sid * rpt, rpt)],
                        out_hbm.at[cid, pl.ds(sid * rpt, rpt)])

    return k(XW, src, dst, gate)


_N_PAD = 10112


def kernel(H, A, E, W1, b1, We1, be1, W2, b2, We2, be2):
    n, d_node = H.shape
    ne = A.shape[1]
    d_edge = E.shape[1]
    n_classes = W2.shape[1]
    d2 = 128

    eye8 = jnp.eye(8, dtype=jnp.float32)
    Wbig = jnp.concatenate(
        [jnp.kron(eye8, We1), jnp.kron(eye8, We2)], axis=1)
    bbig = jnp.concatenate(
        [jnp.tile(be1, 8), jnp.tile(be2, 8)])[None, :]
    E2 = E.reshape(ne // 8, 8 * d_edge)

    g1_8, g2_8 = _tc_gates(E2, Wbig, bbig, block_rows=4000)
    gate1 = g1_8.reshape(ne)
    gate2 = g2_8.reshape(ne)

    src = A[0]
    dst = A[1]
    XW1 = _tc_matmul(H, W1, block_rows=2000)
    parts1 = _sc_aggregate(XW1, src, dst, gate1, _N_PAD)

    W2p = jnp.pad(W2, ((0, 0), (0, d2 - n_classes)))
    XW2 = _tc_mid(parts1, b1[None, :], W2p, block_rows=1264)
    parts2 = _sc_aggregate(XW2, src, dst, gate2, _N_PAD)

    out = _tc_final(parts2, b2[None, :], block_rows=1264)
    return out[:n]


# speedup vs baseline: 5.9778x; 1.2631x over previous
import dataclasses
import functools

import jax
import jax.numpy as jnp
from jax import lax
from jax.experimental import pallas as pl
from jax.experimental.pallas import tpu as pltpu
from jax.experimental.pallas import tpu_sc as plsc

NC = 2
NS = 16
NW = NC * NS
LANES = 16
CHUNK = 80

_SC_CP = pltpu.CompilerParams()
if "needs_layout_passes" in pltpu.CompilerParams.__dataclass_fields__:
    _SC_CP = dataclasses.replace(_SC_CP, needs_layout_passes=False)


def _matmul_kernel(x_ref, w_ref, o_ref):
    o_ref[...] = jnp.dot(x_ref[...], w_ref[...],
                         preferred_element_type=jnp.float32)


def _tc_matmul(X, W, block_rows):
    n, k = X.shape
    m = W.shape[1]
    grid = n // block_rows
    return pl.pallas_call(
        _matmul_kernel,
        grid=(grid,),
        in_specs=[
            pl.BlockSpec((block_rows, k), lambda i: (i, 0)),
            pl.BlockSpec((k, m), lambda i: (0, 0)),
        ],
        out_specs=pl.BlockSpec((block_rows, m), lambda i: (i, 0)),
        out_shape=jax.ShapeDtypeStruct((n, m), jnp.float32),
    )(X, W)


def _gates_kernel(e_ref, w_ref, b_ref, o1_ref, o2_ref):
    logits = jnp.dot(e_ref[...], w_ref[...],
                     preferred_element_type=jnp.float32) + b_ref[...]
    g = jax.nn.sigmoid(logits)
    o1_ref[...] = g[:, :8]
    o2_ref[...] = g[:, 8:]


def _tc_gates(E2, Wbig, bbig, block_rows):
    n = E2.shape[0]
    grid = n // block_rows
    out = jax.ShapeDtypeStruct((n, 8), jnp.float32)
    return pl.pallas_call(
        _gates_kernel,
        grid=(grid,),
        in_specs=[
            pl.BlockSpec((block_rows, 128), lambda i: (i, 0)),
            pl.BlockSpec((128, 16), lambda i: (0, 0)),
            pl.BlockSpec((1, 16), lambda i: (0, 0)),
        ],
        out_specs=[pl.BlockSpec((block_rows, 8), lambda i: (i, 0))] * 2,
        out_shape=[out, out],
    )(E2, Wbig, bbig)


def _mid_kernel(p_ref, b_ref, w_ref, o_ref):
    h = jax.nn.relu(p_ref[0] + p_ref[1] + b_ref[...])
    o_ref[...] = jnp.dot(h, w_ref[...], preferred_element_type=jnp.float32)


def _tc_mid(parts, b1, W2p, block_rows):
    _, n, d = parts.shape
    m = W2p.shape[1]
    grid = n // block_rows
    return pl.pallas_call(
        _mid_kernel,
        grid=(grid,),
        in_specs=[
            pl.BlockSpec((2, block_rows, d), lambda i: (0, i, 0)),
            pl.BlockSpec((1, d), lambda i: (0, 0)),
            pl.BlockSpec((d, m), lambda i: (0, 0)),
        ],
        out_specs=pl.BlockSpec((block_rows, m), lambda i: (i, 0)),
        out_shape=jax.ShapeDtypeStruct((n, m), jnp.float32),
    )(parts, b1, W2p)


def _final_kernel(p_ref, b_ref, o_ref, *, n_classes):
    x = p_ref[0] + p_ref[1]
    logits = x[:, :n_classes] + b_ref[...]
    m = jnp.max(logits, axis=1, keepdims=True)
    s = jnp.log(jnp.sum(jnp.exp(logits - m), axis=1, keepdims=True))
    o_ref[...] = logits - m - s


def _tc_final(parts, b2, block_rows):
    _, n, d = parts.shape
    n_classes = b2.shape[1]
    grid = n // block_rows
    return pl.pallas_call(
        functools.partial(_final_kernel, n_classes=n_classes),
        grid=(grid,),
        in_specs=[
            pl.BlockSpec((2, block_rows, d), lambda i: (0, i, 0)),
            pl.BlockSpec((1, n_classes), lambda i: (0, 0)),
        ],
        out_specs=pl.BlockSpec((block_rows, n_classes), lambda i: (i, 0)),
        out_shape=jax.ShapeDtypeStruct((n, n_classes), jnp.float32),
    )(parts, b2)


def _sc_aggregate(XW, combo, layer, n_out):
    d = XW.shape[1]
    tot_chunks = combo.shape[0]
    nct = tot_chunks // NW
    npairs = nct // 2
    rpt = n_out // NS
    zrows = rpt // 8
    goff = CHUNK * layer

    @functools.partial(
        pl.kernel,
        out_type=jax.ShapeDtypeStruct((NC, n_out, d), jnp.float32),
        mesh=plsc.VectorSubcoreMesh(core_axis_name="c", subcore_axis_name="s"),
        compiler_params=_SC_CP,
        scratch_types=[
            pltpu.VMEM((2, 256), jnp.int32),
            pltpu.VMEM((2, 256), jnp.int32),
            pltpu.VMEM((CHUNK,), jnp.int32),
            pltpu.VMEM((CHUNK,), jnp.int32),
            pltpu.VMEM((CHUNK, d), jnp.float32),
            pltpu.VMEM((CHUNK, d), jnp.float32),
            pltpu.VMEM((zrows, d), jnp.float32),
            pltpu.VMEM_SHARED((n_out, d), jnp.float32),
            pltpu.SemaphoreType.DMA,
            pltpu.SemaphoreType.DMA,
            pltpu.SemaphoreType.DMA,
            pltpu.SemaphoreType.DMA,
            pltpu.SemaphoreType.DMA,
            pltpu.SemaphoreType.DMA,
        ],
    )
    def k(xw_hbm, cb_hbm, out_hbm, cb0, cb1, dstc0, dstc1, rows0, rows1,
          zero_v, acc_sh, semi0, semi1, sem0, sem1, sems0, sems1):
        cid = lax.axis_index("c")
        sid = lax.axis_index("s")
        wid = sid * NC + cid
        base_c = wid * nct

        def idx_start(c, cb_b, semi):
            pltpu.async_copy(cb_hbm.at[base_c + c], cb_b, semi)

        def idx_wait(c, cb_b, semi):
            pltpu.make_async_copy(cb_hbm.at[base_c + c], cb_b, semi).wait()

        def repack(cb_b, dstc_b):
            for kk in range(0, CHUNK, LANES):
                dstc_b[pl.ds(kk, LANES)] = cb_b[0, pl.ds(CHUNK + kk, LANES)]

        def gather_start(cb_b, rows_b, sem):
            pltpu.async_copy(
                xw_hbm.at[cb_b.at[0, pl.ds(0, CHUNK)]], rows_b, sem)

        def gather_wait(cb_b, rows_b, sem):
            pltpu.make_async_copy(
                xw_hbm.at[cb_b.at[0, pl.ds(0, CHUNK)]], rows_b, sem).wait()

        def scatter_start(rows_b, dstc_b, sem):
            pltpu.async_copy(rows_b, acc_sh.at[dstc_b], sem, add=True)

        def scatter_wait(rows_b, dstc_b, sem):
            pltpu.make_async_copy(rows_b, acc_sh.at[dstc_b], sem).wait()

        def scale(rows_b, cb_b):
            @plsc.parallel_loop(0, CHUNK, step=LANES, unroll=2)
            def _(eg):
                gvec = plsc.bitcast(cb_b[1, pl.ds(goff + eg, LANES)],
                                    jnp.float32)
                for j in range(LANES):
                    gb = gvec[jnp.full((LANES,), j, jnp.int32)]
                    for f in range(0, d, LANES):
                        rows_b[eg + j, pl.ds(f, LANES)] = (
                            rows_b[eg + j, pl.ds(f, LANES)] * gb)

        idx_start(0, cb0, semi0)
        idx_start(1, cb1, semi1)
        idx_wait(0, cb0, semi0)
        repack(cb0, dstc0)
        gather_start(cb0, rows0, sem0)
        idx_wait(1, cb1, semi1)
        repack(cb1, dstc1)
        gather_start(cb1, rows1, sem1)

        @pl.loop(0, zrows)
        def _(r):
            for f in range(0, d, LANES):
                zero_v[r, pl.ds(f, LANES)] = jnp.zeros((LANES,), jnp.float32)

        @pl.loop(0, rpt, step=zrows)
        def _(r0):
            pltpu.sync_copy(zero_v, acc_sh.at[pl.ds(sid * rpt + r0, zrows)])

        plsc.subcore_barrier()

        @pl.loop(0, npairs)
        def _(t):
            gather_wait(cb0, rows0, sem0)
            scale(rows0, cb0)
            scatter_start(rows0, dstc0, sems0)

            @pl.when(2 * t + 2 < nct)
            def _():
                idx_start(2 * t + 2, cb0, semi0)

            gather_wait(cb1, rows1, sem1)
            scale(rows1, cb1)
            scatter_start(rows1, dstc1, sems1)

            @pl.when(2 * t + 3 < nct)
            def _():
                idx_start(2 * t + 3, cb1, semi1)

            scatter_wait(rows0, dstc0, sems0)

            @pl.when(2 * t + 2 < nct)
            def _():
                idx_wait(2 * t + 2, cb0, semi0)
                repack(cb0, dstc0)
                gather_start(cb0, rows0, sem0)

            scatter_wait(rows1, dstc1, sems1)

            @pl.when(2 * t + 3 < nct)
            def _():
                idx_wait(2 * t + 3, cb1, semi1)
                repack(cb1, dstc1)
                gather_start(cb1, rows1, sem1)

        if nct % 2 == 1:
            gather_wait(cb0, rows0, sem0)
            scale(rows0, cb0)
            scatter_start(rows0, dstc0, sems0)
            scatter_wait(rows0, dstc0, sems0)

        plsc.subcore_barrier()

        pltpu.sync_copy(acc_sh.at[pl.ds(sid * rpt, rpt)],
                        out_hbm.at[cid, pl.ds(sid * rpt, rpt)])

    return k(XW, combo)


_N_PAD = 10112


def kernel(H, A, E, W1, b1, We1, be1, W2, b2, We2, be2):
    n, d_node = H.shape
    ne = A.shape[1]
    d_edge = E.shape[1]
    n_classes = W2.shape[1]
    d2 = 128

    eye8 = jnp.eye(8, dtype=jnp.float32)
    Wbig = jnp.concatenate(
        [jnp.kron(eye8, We1), jnp.kron(eye8, We2)], axis=1)
    bbig = jnp.concatenate(
        [jnp.tile(be1, 8), jnp.tile(be2, 8)])[None, :]
    E2 = E.reshape(ne // 8, 8 * d_edge)

    g1_8, g2_8 = _tc_gates(E2, Wbig, bbig, block_rows=4000)

    tc = ne // CHUNK
    pad = jnp.zeros((tc, 256 - 2 * CHUNK), jnp.int32)
    row0 = jnp.concatenate(
        [A[0].reshape(tc, CHUNK), A[1].reshape(tc, CHUNK), pad], axis=1)
    row1 = jnp.concatenate(
        [jax.lax.bitcast_convert_type(g1_8.reshape(tc, CHUNK), jnp.int32),
         jax.lax.bitcast_convert_type(g2_8.reshape(tc, CHUNK), jnp.int32),
         pad], axis=1)
    combo = jnp.stack([row0, row1], axis=1)

    XW1 = _tc_matmul(H, W1, block_rows=2000)
    parts1 = _sc_aggregate(XW1, combo, 0, _N_PAD)

    W2p = jnp.pad(W2, ((0, 0), (0, d2 - n_classes)))
    XW2 = _tc_mid(parts1, b1[None, :], W2p, block_rows=1264)
    parts2 = _sc_aggregate(XW2, combo, 1, _N_PAD)

    out = _tc_final(parts2, b2[None, :], block_rows=1264)
    return out[:n]


# speedup vs baseline: 6.3943x; 1.0697x over previous
import dataclasses
import functools

import jax
import jax.numpy as jnp
from jax import lax
from jax.experimental import pallas as pl
from jax.experimental.pallas import tpu as pltpu
from jax.experimental.pallas import tpu_sc as plsc

NC = 2
NS = 16
NW = NC * NS
LANES = 16
CHUNK = 80

_SC_CP = pltpu.CompilerParams()
if "needs_layout_passes" in pltpu.CompilerParams.__dataclass_fields__:
    _SC_CP = dataclasses.replace(_SC_CP, needs_layout_passes=False)


def _matmul_kernel(x_ref, w_ref, o_ref):
    o_ref[...] = jnp.dot(x_ref[...], w_ref[...],
                         preferred_element_type=jnp.float32)


def _tc_matmul(X, W, block_rows):
    n, k = X.shape
    m = W.shape[1]
    grid = n // block_rows
    return pl.pallas_call(
        _matmul_kernel,
        grid=(grid,),
        in_specs=[
            pl.BlockSpec((block_rows, k), lambda i: (i, 0)),
            pl.BlockSpec((k, m), lambda i: (0, 0)),
        ],
        out_specs=pl.BlockSpec((block_rows, m), lambda i: (i, 0)),
        out_shape=jax.ShapeDtypeStruct((n, m), jnp.float32),
    )(X, W)


def _gates_kernel(e_ref, w_ref, b_ref, o_ref):
    logits = jnp.dot(e_ref[...], w_ref[...],
                     preferred_element_type=jnp.float32) + b_ref[...]
    o_ref[...] = jax.nn.sigmoid(logits).T


def _tc_gates(E, Wcat, bcat, block_rows):
    n, d_e = E.shape
    grid = n // block_rows
    return pl.pallas_call(
        _gates_kernel,
        grid=(grid,),
        in_specs=[
            pl.BlockSpec((block_rows, d_e), lambda i: (i, 0)),
            pl.BlockSpec((d_e, 2), lambda i: (0, 0)),
            pl.BlockSpec((1, 2), lambda i: (0, 0)),
        ],
        out_specs=pl.BlockSpec((2, block_rows), lambda i: (0, i)),
        out_shape=jax.ShapeDtypeStruct((2, n), jnp.float32),
    )(E, Wcat, bcat)


def _mid_kernel(p_ref, b_ref, w_ref, o_ref):
    h = jax.nn.relu(p_ref[0] + p_ref[1] + b_ref[...])
    o_ref[...] = jnp.dot(h, w_ref[...], preferred_element_type=jnp.float32)


def _tc_mid(parts, b1, W2p, block_rows):
    _, n, d = parts.shape
    m = W2p.shape[1]
    grid = n // block_rows
    return pl.pallas_call(
        _mid_kernel,
        grid=(grid,),
        in_specs=[
            pl.BlockSpec((2, block_rows, d), lambda i: (0, i, 0)),
            pl.BlockSpec((1, d), lambda i: (0, 0)),
            pl.BlockSpec((d, m), lambda i: (0, 0)),
        ],
        out_specs=pl.BlockSpec((block_rows, m), lambda i: (i, 0)),
        out_shape=jax.ShapeDtypeStruct((n, m), jnp.float32),
    )(parts, b1, W2p)


def _final_kernel(p_ref, b_ref, o_ref, *, n_classes):
    x = p_ref[0] + p_ref[1]
    logits = x[:, :n_classes] + b_ref[...]
    m = jnp.max(logits, axis=1, keepdims=True)
    s = jnp.log(jnp.sum(jnp.exp(logits - m), axis=1, keepdims=True))
    o_ref[...] = logits - m - s


def _tc_final(parts, b2, block_rows):
    _, n, d = parts.shape
    n_classes = b2.shape[1]
    grid = n // block_rows
    return pl.pallas_call(
        functools.partial(_final_kernel, n_classes=n_classes),
        grid=(grid,),
        in_specs=[
            pl.BlockSpec((2, block_rows, d), lambda i: (0, i, 0)),
            pl.BlockSpec((1, n_classes), lambda i: (0, 0)),
        ],
        out_specs=pl.BlockSpec((block_rows, n_classes), lambda i: (i, 0)),
        out_shape=jax.ShapeDtypeStruct((n, n_classes), jnp.float32),
    )(parts, b2)


def _sc_aggregate(XW, row0, row1, layer, n_out):
    d = XW.shape[1]
    tot_chunks = row0.shape[0]
    nct = tot_chunks // NW
    npairs = nct // 2
    rpt = n_out // NS
    zrows = rpt // 8
    goff = CHUNK * layer

    @functools.partial(
        pl.kernel,
        out_type=jax.ShapeDtypeStruct((NC, n_out, d), jnp.float32),
        mesh=plsc.VectorSubcoreMesh(core_axis_name="c", subcore_axis_name="s"),
        compiler_params=_SC_CP,
        scratch_types=[
            pltpu.VMEM((2, 256), jnp.int32),
            pltpu.VMEM((2, 256), jnp.int32),
            pltpu.VMEM((CHUNK,), jnp.int32),
            pltpu.VMEM((CHUNK,), jnp.int32),
            pltpu.VMEM((CHUNK, d), jnp.float32),
            pltpu.VMEM((CHUNK, d), jnp.float32),
            pltpu.VMEM((zrows, d), jnp.float32),
            pltpu.VMEM_SHARED((n_out, d), jnp.float32),
            pltpu.SemaphoreType.DMA,
            pltpu.SemaphoreType.DMA,
            pltpu.SemaphoreType.DMA,
            pltpu.SemaphoreType.DMA,
            pltpu.SemaphoreType.DMA,
            pltpu.SemaphoreType.DMA,
        ],
    )
    def k(xw_hbm, r0_hbm, r1_hbm, out_hbm, cb0, cb1, dstc0, dstc1, rows0,
          rows1, zero_v, acc_sh, semi0, semi1, sem0, sem1, sems0, sems1):
        cid = lax.axis_index("c")
        sid = lax.axis_index("s")
        wid = sid * NC + cid
        base_c = wid * nct

        def idx_start(c, cb_b, semi):
            pltpu.async_copy(r0_hbm.at[base_c + c], cb_b.at[0], semi)
            pltpu.async_copy(r1_hbm.at[base_c + c], cb_b.at[1], semi)

        def idx_wait(c, cb_b, semi):
            pltpu.make_async_copy(r0_hbm.at[base_c + c], cb_b.at[0],
                                  semi).wait()
            pltpu.make_async_copy(r1_hbm.at[base_c + c], cb_b.at[1],
                                  semi).wait()

        def repack(cb_b, dstc_b):
            for kk in range(0, CHUNK, LANES):
                dstc_b[pl.ds(kk, LANES)] = cb_b[0, pl.ds(CHUNK + kk, LANES)]

        def gather_start(cb_b, rows_b, sem):
            pltpu.async_copy(
                xw_hbm.at[cb_b.at[0, pl.ds(0, CHUNK)]], rows_b, sem)

        def gather_wait(cb_b, rows_b, sem):
            pltpu.make_async_copy(
                xw_hbm.at[cb_b.at[0, pl.ds(0, CHUNK)]], rows_b, sem).wait()

        def scatter_start(rows_b, dstc_b, sem):
            pltpu.async_copy(rows_b, acc_sh.at[dstc_b], sem, add=True)

        def scatter_wait(rows_b, dstc_b, sem):
            pltpu.make_async_copy(rows_b, acc_sh.at[dstc_b], sem).wait()

        def scale(rows_b, cb_b):
            @plsc.parallel_loop(0, CHUNK, step=LANES, unroll=2)
            def _(eg):
                gvec = plsc.bitcast(cb_b[1, pl.ds(goff + eg, LANES)],
                                    jnp.float32)
                for j in range(LANES):
                    gb = gvec[jnp.full((LANES,), j, jnp.int32)]
                    for f in range(0, d, LANES):
                        rows_b[eg + j, pl.ds(f, LANES)] = (
                            rows_b[eg + j, pl.ds(f, LANES)] * gb)

        idx_start(0, cb0, semi0)
        idx_start(1, cb1, semi1)
        idx_wait(0, cb0, semi0)
        repack(cb0, dstc0)
        gather_start(cb0, rows0, sem0)
        idx_wait(1, cb1, semi1)
        repack(cb1, dstc1)
        gather_start(cb1, rows1, sem1)

        @pl.loop(0, zrows)
        def _(r):
            for f in range(0, d, LANES):
                zero_v[r, pl.ds(f, LANES)] = jnp.zeros((LANES,), jnp.float32)

        @pl.loop(0, rpt, step=zrows)
        def _(r0):
            pltpu.sync_copy(zero_v, acc_sh.at[pl.ds(sid * rpt + r0, zrows)])

        plsc.subcore_barrier()

        @pl.loop(0, npairs)
        def _(t):
            gather_wait(cb0, rows0, sem0)
            scale(rows0, cb0)
            scatter_start(rows0, dstc0, sems0)

            @pl.when(2 * t + 2 < nct)
            def _():
                idx_start(2 * t + 2, cb0, semi0)

            gather_wait(cb1, rows1, sem1)
            scale(rows1, cb1)
            scatter_start(rows1, dstc1, sems1)

            @pl.when(2 * t + 3 < nct)
            def _():
                idx_start(2 * t + 3, cb1, semi1)

            scatter_wait(rows0, dstc0, sems0)

            @pl.when(2 * t + 2 < nct)
            def _():
                idx_wait(2 * t + 2, cb0, semi0)
                repack(cb0, dstc0)
                gather_start(cb0, rows0, sem0)

            scatter_wait(rows1, dstc1, sems1)

            @pl.when(2 * t + 3 < nct)
            def _():
                idx_wait(2 * t + 3, cb1, semi1)
                repack(cb1, dstc1)
                gather_start(cb1, rows1, sem1)

        if nct % 2 == 1:
            gather_wait(cb0, rows0, sem0)
            scale(rows0, cb0)
            scatter_start(rows0, dstc0, sems0)
            scatter_wait(rows0, dstc0, sems0)

        plsc.subcore_barrier()

        pltpu.sync_copy(acc_sh.at[pl.ds(sid * rpt, rpt)],
                        out_hbm.at[cid, pl.ds(sid * rpt, rpt)])

    return k(XW, row0, row1)


_N_PAD = 10112


def kernel(H, A, E, W1, b1, We1, be1, W2, b2, We2, be2):
    n, d_node = H.shape
    ne = A.shape[1]
    d_edge = E.shape[1]
    n_classes = W2.shape[1]
    d2 = 128

    Wcat = jnp.concatenate([We1, We2], axis=1)
    bcat = jnp.concatenate([be1, be2])[None, :]
    gates_t = _tc_gates(E, Wcat, bcat, block_rows=12800)

    tc = ne // CHUNK
    pad = jnp.zeros((tc, 256 - 2 * CHUNK), jnp.int32)
    row0 = jnp.concatenate(
        [A[0].reshape(tc, CHUNK), A[1].reshape(tc, CHUNK), pad], axis=1)
    gbits = jax.lax.bitcast_convert_type(gates_t, jnp.int32)
    row1 = jnp.concatenate(
        [gbits[0].reshape(tc, CHUNK), gbits[1].reshape(tc, CHUNK), pad],
        axis=1)

    XW1 = _tc_matmul(H, W1, block_rows=2000)
    parts1 = _sc_aggregate(XW1, row0, row1, 0, _N_PAD)

    W2p = jnp.pad(W2, ((0, 0), (0, d2 - n_classes)))
    XW2 = _tc_mid(parts1, b1[None, :], W2p, block_rows=1264)
    parts2 = _sc_aggregate(XW2, row0, row1, 1, _N_PAD)

    out = _tc_final(parts2, b2[None, :], block_rows=1264)
    return out[:n]


# speedup vs baseline: 8.2762x; 1.2943x over previous
import dataclasses
import functools

import jax
import jax.numpy as jnp
from jax import lax
from jax.experimental import pallas as pl
from jax.experimental.pallas import tpu as pltpu
from jax.experimental.pallas import tpu_sc as plsc

NC = 2
NS = 16
NW = NC * NS
LANES = 16
CHUNK = 80

_SC_CP = pltpu.CompilerParams()
if "needs_layout_passes" in pltpu.CompilerParams.__dataclass_fields__:
    _SC_CP = dataclasses.replace(_SC_CP, needs_layout_passes=False)


def _matmul_kernel(x_ref, w_ref, o_ref):
    o_ref[...] = jnp.dot(x_ref[...], w_ref[...],
                         preferred_element_type=jnp.float32)


def _tc_matmul(X, W, block_rows):
    n, k = X.shape
    m = W.shape[1]
    grid = n // block_rows
    return pl.pallas_call(
        _matmul_kernel,
        grid=(grid,),
        in_specs=[
            pl.BlockSpec((block_rows, k), lambda i: (i, 0)),
            pl.BlockSpec((k, m), lambda i: (0, 0)),
        ],
        out_specs=pl.BlockSpec((block_rows, m), lambda i: (i, 0)),
        out_shape=jax.ShapeDtypeStruct((n, m), jnp.float32),
    )(X, W)


def _gates_kernel(e_ref, w_ref, b_ref, o_ref):
    logits = jnp.dot(w_ref[...], e_ref[...],
                     preferred_element_type=jnp.float32) + b_ref[...]
    o_ref[...] = jax.nn.sigmoid(logits)


def _tc_gates(ET, WcatT, bcatT, block_cols):
    d_e, n = ET.shape
    grid = n // block_cols
    return pl.pallas_call(
        _gates_kernel,
        grid=(grid,),
        in_specs=[
            pl.BlockSpec((d_e, block_cols), lambda i: (0, i)),
            pl.BlockSpec((2, d_e), lambda i: (0, 0)),
            pl.BlockSpec((2, 1), lambda i: (0, 0)),
        ],
        out_specs=pl.BlockSpec((2, block_cols), lambda i: (0, i)),
        out_shape=jax.ShapeDtypeStruct((2, n), jnp.float32),
    )(ET, WcatT, bcatT)


def _mid_kernel(p_ref, b_ref, w_ref, o_ref):
    h = jax.nn.relu(p_ref[0] + p_ref[1] + b_ref[...])
    o_ref[...] = jnp.dot(h, w_ref[...], preferred_element_type=jnp.float32)


def _tc_mid(parts, b1, W2p, block_rows):
    _, n, d = parts.shape
    m = W2p.shape[1]
    grid = n // block_rows
    return pl.pallas_call(
        _mid_kernel,
        grid=(grid,),
        in_specs=[
            pl.BlockSpec((2, block_rows, d), lambda i: (0, i, 0)),
            pl.BlockSpec((1, d), lambda i: (0, 0)),
            pl.BlockSpec((d, m), lambda i: (0, 0)),
        ],
        out_specs=pl.BlockSpec((block_rows, m), lambda i: (i, 0)),
        out_shape=jax.ShapeDtypeStruct((n, m), jnp.float32),
    )(parts, b1, W2p)


def _final_kernel(p_ref, b_ref, o_ref, *, n_classes):
    x = p_ref[0] + p_ref[1]
    logits = x[:, :n_classes] + b_ref[...]
    m = jnp.max(logits, axis=1, keepdims=True)
    s = jnp.log(jnp.sum(jnp.exp(logits - m), axis=1, keepdims=True))
    o_ref[...] = logits - m - s


def _tc_final(parts, b2, block_rows):
    _, n, d = parts.shape
    n_classes = b2.shape[1]
    grid = n // block_rows
    return pl.pallas_call(
        functools.partial(_final_kernel, n_classes=n_classes),
        grid=(grid,),
        in_specs=[
            pl.BlockSpec((2, block_rows, d), lambda i: (0, i, 0)),
            pl.BlockSpec((1, n_classes), lambda i: (0, 0)),
        ],
        out_specs=pl.BlockSpec((block_rows, n_classes), lambda i: (i, 0)),
        out_shape=jax.ShapeDtypeStruct((n, n_classes), jnp.float32),
    )(parts, b2)


def _sc_aggregate(XW, row0, row1, layer, n_out):
    d = XW.shape[1]
    tot_chunks = row0.shape[0]
    nct = tot_chunks // NW
    npairs = nct // 2
    rpt = n_out // NS
    zrows = rpt // 8
    goff = CHUNK * layer

    @functools.partial(
        pl.kernel,
        out_type=jax.ShapeDtypeStruct((NC, n_out, d), jnp.float32),
        mesh=plsc.VectorSubcoreMesh(core_axis_name="c", subcore_axis_name="s"),
        compiler_params=_SC_CP,
        scratch_types=[
            pltpu.VMEM((2, 256), jnp.int32),
            pltpu.VMEM((2, 256), jnp.int32),
            pltpu.VMEM((CHUNK,), jnp.int32),
            pltpu.VMEM((CHUNK,), jnp.int32),
            pltpu.VMEM((CHUNK, d), jnp.float32),
            pltpu.VMEM((CHUNK, d), jnp.float32),
            pltpu.VMEM((zrows, d), jnp.float32),
            pltpu.VMEM_SHARED((n_out, d), jnp.float32),
            pltpu.SemaphoreType.DMA,
            pltpu.SemaphoreType.DMA,
            pltpu.SemaphoreType.DMA,
            pltpu.SemaphoreType.DMA,
            pltpu.SemaphoreType.DMA,
            pltpu.SemaphoreType.DMA,
        ],
    )
    def k(xw_hbm, r0_hbm, r1_hbm, out_hbm, cb0, cb1, dstc0, dstc1, rows0,
          rows1, zero_v, acc_sh, semi0, semi1, sem0, sem1, sems0, sems1):
        cid = lax.axis_index("c")
        sid = lax.axis_index("s")
        wid = sid * NC + cid
        base_c = wid * nct

        def idx_start(c, cb_b, semi):
            pltpu.async_copy(r0_hbm.at[base_c + c], cb_b.at[0], semi)
            pltpu.async_copy(r1_hbm.at[base_c + c], cb_b.at[1], semi)

        def idx_wait(c, cb_b, semi):
            pltpu.make_async_copy(r0_hbm.at[base_c + c], cb_b.at[0],
                                  semi).wait()
            pltpu.make_async_copy(r1_hbm.at[base_c + c], cb_b.at[1],
                                  semi).wait()

        def repack(cb_b, dstc_b):
            for kk in range(0, CHUNK, LANES):
                dstc_b[pl.ds(kk, LANES)] = cb_b[0, pl.ds(CHUNK + kk, LANES)]

        def gather_start(cb_b, rows_b, sem):
            pltpu.async_copy(
                xw_hbm.at[cb_b.at[0, pl.ds(0, CHUNK)]], rows_b, sem)

        def gather_wait(cb_b, rows_b, sem):
            pltpu.make_async_copy(
                xw_hbm.at[cb_b.at[0, pl.ds(0, CHUNK)]], rows_b, sem).wait()

        def scatter_start(rows_b, dstc_b, sem):
            pltpu.async_copy(rows_b, acc_sh.at[dstc_b], sem, add=True)

        def scatter_wait(rows_b, dstc_b, sem):
            pltpu.make_async_copy(rows_b, acc_sh.at[dstc_b], sem).wait()

        def scale(rows_b, cb_b):
            @plsc.parallel_loop(0, CHUNK, step=LANES, unroll=2)
            def _(eg):
                gvec = plsc.bitcast(cb_b[1, pl.ds(goff + eg, LANES)],
                                    jnp.float32)
                for j in range(LANES):
                    gb = gvec[jnp.full((LANES,), j, jnp.int32)]
                    for f in range(0, d, LANES):
                        rows_b[eg + j, pl.ds(f, LANES)] = (
                            rows_b[eg + j, pl.ds(f, LANES)] * gb)

        idx_start(0, cb0, semi0)
        idx_start(1, cb1, semi1)
        idx_wait(0, cb0, semi0)
        repack(cb0, dstc0)
        gather_start(cb0, rows0, sem0)
        idx_wait(1, cb1, semi1)
        repack(cb1, dstc1)
        gather_start(cb1, rows1, sem1)

        @pl.loop(0, zrows)
        def _(r):
            for f in range(0, d, LANES):
                zero_v[r, pl.ds(f, LANES)] = jnp.zeros((LANES,), jnp.float32)

        @pl.loop(0, rpt, step=zrows)
        def _(r0):
            pltpu.sync_copy(zero_v, acc_sh.at[pl.ds(sid * rpt + r0, zrows)])

        plsc.subcore_barrier()

        @pl.loop(0, npairs)
        def _(t):
            gather_wait(cb0, rows0, sem0)
            scale(rows0, cb0)
            scatter_start(rows0, dstc0, sems0)

            @pl.when(2 * t + 2 < nct)
            def _():
                idx_start(2 * t + 2, cb0, semi0)

            gather_wait(cb1, rows1, sem1)
            scale(rows1, cb1)
            scatter_start(rows1, dstc1, sems1)

            @pl.when(2 * t + 3 < nct)
            def _():
                idx_start(2 * t + 3, cb1, semi1)

            scatter_wait(rows0, dstc0, sems0)

            @pl.when(2 * t + 2 < nct)
            def _():
                idx_wait(2 * t + 2, cb0, semi0)
                repack(cb0, dstc0)
                gather_start(cb0, rows0, sem0)

            scatter_wait(rows1, dstc1, sems1)

            @pl.when(2 * t + 3 < nct)
            def _():
                idx_wait(2 * t + 3, cb1, semi1)
                repack(cb1, dstc1)
                gather_start(cb1, rows1, sem1)

        if nct % 2 == 1:
            gather_wait(cb0, rows0, sem0)
            scale(rows0, cb0)
            scatter_start(rows0, dstc0, sems0)
            scatter_wait(rows0, dstc0, sems0)

        plsc.subcore_barrier()

        pltpu.sync_copy(acc_sh.at[pl.ds(sid * rpt, rpt)],
                        out_hbm.at[cid, pl.ds(sid * rpt, rpt)])

    return k(XW, row0, row1)


_N_PAD = 10112


def kernel(H, A, E, W1, b1, We1, be1, W2, b2, We2, be2):
    n, d_node = H.shape
    ne = A.shape[1]
    d_edge = E.shape[1]
    n_classes = W2.shape[1]
    d2 = 128

    WcatT = jnp.concatenate([We1, We2], axis=1).T
    bcatT = jnp.concatenate([be1, be2])[:, None]
    gates_t = _tc_gates(E.T, WcatT, bcatT, block_cols=12800)

    tc = ne // CHUNK
    pad = jnp.zeros((tc, 256 - 2 * CHUNK), jnp.int32)
    row0 = jnp.concatenate(
        [A[0].reshape(tc, CHUNK), A[1].reshape(tc, CHUNK), pad], axis=1)
    gbits = jax.lax.bitcast_convert_type(gates_t, jnp.int32)
    row1 = jnp.concatenate(
        [gbits[0].reshape(tc, CHUNK), gbits[1].reshape(tc, CHUNK), pad],
        axis=1)

    XW1 = _tc_matmul(H, W1, block_rows=2000)
    parts1 = _sc_aggregate(XW1, row0, row1, 0, _N_PAD)

    W2p = jnp.pad(W2, ((0, 0), (0, d2 - n_classes)))
    XW2 = _tc_mid(parts1, b1[None, :], W2p, block_rows=1264)
    parts2 = _sc_aggregate(XW2, row0, row1, 1, _N_PAD)

    out = _tc_final(parts2, b2[None, :], block_rows=1264)
    return out[:n]


# speedup vs baseline: 8.8886x; 1.0740x over previous
import dataclasses
import functools

import jax
import jax.numpy as jnp
from jax import lax
from jax.experimental import pallas as pl
from jax.experimental.pallas import tpu as pltpu
from jax.experimental.pallas import tpu_sc as plsc

NC = 2
NS = 16
NW = NC * NS
LANES = 16
CHUNK = 128

_SC_CP = pltpu.CompilerParams()
if "needs_layout_passes" in pltpu.CompilerParams.__dataclass_fields__:
    _SC_CP = dataclasses.replace(_SC_CP, needs_layout_passes=False)


def _matmul_kernel(x_ref, w_ref, o_ref):
    o_ref[...] = jnp.dot(x_ref[...], w_ref[...],
                         preferred_element_type=jnp.float32)


def _tc_matmul(X, W, block_rows):
    n, k = X.shape
    m = W.shape[1]
    grid = n // block_rows
    return pl.pallas_call(
        _matmul_kernel,
        grid=(grid,),
        in_specs=[
            pl.BlockSpec((block_rows, k), lambda i: (i, 0)),
            pl.BlockSpec((k, m), lambda i: (0, 0)),
        ],
        out_specs=pl.BlockSpec((block_rows, m), lambda i: (i, 0)),
        out_shape=jax.ShapeDtypeStruct((n, m), jnp.float32),
    )(X, W)


def _gates_kernel(e_ref, w_ref, b_ref, o_ref):
    logits = jnp.dot(w_ref[...], e_ref[...],
                     preferred_element_type=jnp.float32) + b_ref[...]
    o_ref[...] = jax.nn.sigmoid(logits)


def _tc_gates(ET, WcatT, bcatT, block_cols):
    d_e, n = ET.shape
    grid = n // block_cols
    return pl.pallas_call(
        _gates_kernel,
        grid=(grid,),
        in_specs=[
            pl.BlockSpec((d_e, block_cols), lambda i: (0, i)),
            pl.BlockSpec((2, d_e), lambda i: (0, 0)),
            pl.BlockSpec((2, 1), lambda i: (0, 0)),
        ],
        out_specs=pl.BlockSpec((2, block_cols), lambda i: (0, i)),
        out_shape=jax.ShapeDtypeStruct((2, n), jnp.float32),
    )(ET, WcatT, bcatT)


def _mid_kernel(p_ref, b_ref, w_ref, o_ref):
    h = jax.nn.relu(p_ref[0] + p_ref[1] + b_ref[...])
    o_ref[...] = jnp.dot(h, w_ref[...], preferred_element_type=jnp.float32)


def _tc_mid(parts, b1, W2p, block_rows):
    _, n, d = parts.shape
    m = W2p.shape[1]
    grid = n // block_rows
    return pl.pallas_call(
        _mid_kernel,
        grid=(grid,),
        in_specs=[
            pl.BlockSpec((2, block_rows, d), lambda i: (0, i, 0)),
            pl.BlockSpec((1, d), lambda i: (0, 0)),
            pl.BlockSpec((d, m), lambda i: (0, 0)),
        ],
        out_specs=pl.BlockSpec((block_rows, m), lambda i: (i, 0)),
        out_shape=jax.ShapeDtypeStruct((n, m), jnp.float32),
    )(parts, b1, W2p)


def _final_kernel(p_ref, b_ref, o_ref, *, n_classes):
    x = p_ref[0] + p_ref[1]
    logits = x[:, :n_classes] + b_ref[...]
    m = jnp.max(logits, axis=1, keepdims=True)
    s = jnp.log(jnp.sum(jnp.exp(logits - m), axis=1, keepdims=True))
    o_ref[...] = logits - m - s


def _tc_final(parts, b2, block_rows):
    _, n, d = parts.shape
    n_classes = b2.shape[1]
    grid = n // block_rows
    return pl.pallas_call(
        functools.partial(_final_kernel, n_classes=n_classes),
        grid=(grid,),
        in_specs=[
            pl.BlockSpec((2, block_rows, d), lambda i: (0, i, 0)),
            pl.BlockSpec((1, n_classes), lambda i: (0, 0)),
        ],
        out_specs=pl.BlockSpec((block_rows, n_classes), lambda i: (i, 0)),
        out_shape=jax.ShapeDtypeStruct((n, n_classes), jnp.float32),
    )(parts, b2)


def _sc_aggregate(XW, row0, row1, layer, n_out):
    d = XW.shape[1]
    tot_chunks = row0.shape[0]
    base_ct = tot_chunks // NW
    rem = tot_chunks % NW
    npairs = base_ct // 2
    rpt = n_out // NS
    zrows = rpt // 8
    goff = CHUNK * layer

    @functools.partial(
        pl.kernel,
        out_type=jax.ShapeDtypeStruct((NC, n_out, d), jnp.float32),
        mesh=plsc.VectorSubcoreMesh(core_axis_name="c", subcore_axis_name="s"),
        compiler_params=_SC_CP,
        scratch_types=[
            pltpu.VMEM((2, 256), jnp.int32),
            pltpu.VMEM((2, 256), jnp.int32),
            pltpu.VMEM((CHUNK,), jnp.int32),
            pltpu.VMEM((CHUNK,), jnp.int32),
            pltpu.VMEM((CHUNK, d), jnp.float32),
            pltpu.VMEM((CHUNK, d), jnp.float32),
            pltpu.VMEM((zrows, d), jnp.float32),
            pltpu.VMEM_SHARED((n_out, d), jnp.float32),
            pltpu.SemaphoreType.DMA,
            pltpu.SemaphoreType.DMA,
            pltpu.SemaphoreType.DMA,
            pltpu.SemaphoreType.DMA,
            pltpu.SemaphoreType.DMA,
            pltpu.SemaphoreType.DMA,
        ],
    )
    def k(xw_hbm, r0_hbm, r1_hbm, out_hbm, cb0, cb1, dstc0, dstc1, rows0,
          rows1, zero_v, acc_sh, semi0, semi1, sem0, sem1, sems0, sems1):
        cid = lax.axis_index("c")
        sid = lax.axis_index("s")
        wid = sid * NC + cid
        nct = base_ct + jnp.where(wid < rem, 1, 0)
        base_c = wid * base_ct + jnp.minimum(wid, rem)

        def idx_start(c, cb_b, semi):
            pltpu.async_copy(r0_hbm.at[base_c + c], cb_b.at[0], semi)
            pltpu.async_copy(r1_hbm.at[base_c + c], cb_b.at[1], semi)

        def idx_wait(c, cb_b, semi):
            pltpu.make_async_copy(r0_hbm.at[base_c + c], cb_b.at[0],
                                  semi).wait()
            pltpu.make_async_copy(r1_hbm.at[base_c + c], cb_b.at[1],
                                  semi).wait()

        def repack(cb_b, dstc_b):
            for kk in range(0, CHUNK, LANES):
                dstc_b[pl.ds(kk, LANES)] = cb_b[0, pl.ds(CHUNK + kk, LANES)]

        def gather_start(cb_b, rows_b, sem):
            pltpu.async_copy(
                xw_hbm.at[cb_b.at[0, pl.ds(0, CHUNK)]], rows_b, sem)

        def gather_wait(cb_b, rows_b, sem):
            pltpu.make_async_copy(
                xw_hbm.at[cb_b.at[0, pl.ds(0, CHUNK)]], rows_b, sem).wait()

        def scatter_start(rows_b, dstc_b, sem):
            pltpu.async_copy(rows_b, acc_sh.at[dstc_b], sem, add=True)

        def scatter_wait(rows_b, dstc_b, sem):
            pltpu.make_async_copy(rows_b, acc_sh.at[dstc_b], sem).wait()

        def scale(rows_b, cb_b):
            @plsc.parallel_loop(0, CHUNK, step=LANES, unroll=2)
            def _(eg):
                gvec = plsc.bitcast(cb_b[1, pl.ds(goff + eg, LANES)],
                                    jnp.float32)
                for j in range(LANES):
                    gb = gvec[jnp.full((LANES,), j, jnp.int32)]
                    for f in range(0, d, LANES):
                        rows_b[eg + j, pl.ds(f, LANES)] = (
                            rows_b[eg + j, pl.ds(f, LANES)] * gb)

        idx_start(0, cb0, semi0)
        idx_start(1, cb1, semi1)
        idx_wait(0, cb0, semi0)
        repack(cb0, dstc0)
        gather_start(cb0, rows0, sem0)
        idx_wait(1, cb1, semi1)
        repack(cb1, dstc1)
        gather_start(cb1, rows1, sem1)

        @pl.loop(0, zrows)
        def _(r):
            for f in range(0, d, LANES):
                zero_v[r, pl.ds(f, LANES)] = jnp.zeros((LANES,), jnp.float32)

        @pl.loop(0, rpt, step=zrows)
        def _(r0):
            pltpu.sync_copy(zero_v, acc_sh.at[pl.ds(sid * rpt + r0, zrows)])

        plsc.subcore_barrier()

        @pl.loop(0, npairs)
        def _(t):
            gather_wait(cb0, rows0, sem0)
            scale(rows0, cb0)
            scatter_start(rows0, dstc0, sems0)

            @pl.when(2 * t + 2 < nct)
            def _():
                idx_start(2 * t + 2, cb0, semi0)

            gather_wait(cb1, rows1, sem1)
            scale(rows1, cb1)
            scatter_start(rows1, dstc1, sems1)

            @pl.when(2 * t + 3 < nct)
            def _():
                idx_start(2 * t + 3, cb1, semi1)

            scatter_wait(rows0, dstc0, sems0)

            @pl.when(2 * t + 2 < nct)
            def _():
                idx_wait(2 * t + 2, cb0, semi0)
                repack(cb0, dstc0)
                gather_start(cb0, rows0, sem0)

            scatter_wait(rows1, dstc1, sems1)

            @pl.when(2 * t + 3 < nct)
            def _():
                idx_wait(2 * t + 3, cb1, semi1)
                repack(cb1, dstc1)
                gather_start(cb1, rows1, sem1)

        @pl.when(2 * npairs < nct)
        def _():
            gather_wait(cb0, rows0, sem0)
            scale(rows0, cb0)
            scatter_start(rows0, dstc0, sems0)
            scatter_wait(rows0, dstc0, sems0)

        plsc.subcore_barrier()

        pltpu.sync_copy(acc_sh.at[pl.ds(sid * rpt, rpt)],
                        out_hbm.at[cid, pl.ds(sid * rpt, rpt)])

    return k(XW, row0, row1)


_N_PAD = 10112


def kernel(H, A, E, W1, b1, We1, be1, W2, b2, We2, be2):
    n, d_node = H.shape
    ne = A.shape[1]
    d_edge = E.shape[1]
    n_classes = W2.shape[1]
    d2 = 128

    WcatT = jnp.concatenate([We1, We2], axis=1).T
    bcatT = jnp.concatenate([be1, be2])[:, None]
    gates_t = _tc_gates(E.T, WcatT, bcatT, block_cols=12800)

    tc = ne // CHUNK
    row0 = [A[0].reshape(tc, CHUNK), A[1].reshape(tc, CHUNK)]
    gbits = jax.lax.bitcast_convert_type(gates_t, jnp.int32)
    row1 = [gbits[0].reshape(tc, CHUNK), gbits[1].reshape(tc, CHUNK)]
    if 2 * CHUNK < 256:
        pad = jnp.zeros((tc, 256 - 2 * CHUNK), jnp.int32)
        row0.append(pad)
        row1.append(pad)
    row0 = jnp.concatenate(row0, axis=1)
    row1 = jnp.concatenate(row1, axis=1)

    XW1 = _tc_matmul(H, W1, block_rows=2000)
    parts1 = _sc_aggregate(XW1, row0, row1, 0, _N_PAD)

    W2p = jnp.pad(W2, ((0, 0), (0, d2 - n_classes)))
    XW2 = _tc_mid(parts1, b1[None, :], W2p, block_rows=1264)
    parts2 = _sc_aggregate(XW2, row0, row1, 1, _N_PAD)

    out = _tc_final(parts2, b2[None, :], block_rows=1264)
    return out[:n]
